# baseline (speedup 1.0000x reference)
import jax
import jax.numpy as jnp
from jax import lax
from jax.experimental import pallas as pl
from jax.experimental.pallas import tpu as pltpu
from jax.experimental.pallas import tpu_sc as plsc

_NCLS = 31
_BG = 30
_BUDGET = 80000.0
_H = 512
_W = 512
_ROWS = 128
_TAIL_B = 3
_NW = 32
_RPW = _H // _NW
_TR = 8
_TC = 128
_CGS = _W // _TC
_NCH = (_RPW // _TR) * _CGS
_LANES = 16


def _head_tc(x_ref, t_ref, out_ref, acc_ref, cnt_ref):
    b = pl.program_id(0)
    r = pl.program_id(1)
    nb = pl.num_programs(1)
    step = b * nb + r
    nsteps = pl.num_programs(0) * nb

    @pl.when(step == 0)
    def _init():
        acc_ref[0] = 0.0
        acc_ref[1] = 0.0
        cnt_ref[0] = 0

    x = x_ref[0]
    m = x[0]
    idx = jnp.zeros_like(m)
    for c in range(1, _NCLS):
        xc = x[c]
        gt = xc > m
        m = jnp.where(gt, xc, m)
        idx = jnp.where(gt, jnp.float32(c), idx)
    p = idx

    t = t_ref[0]
    tf = t.astype(jnp.float32)
    bg = t == _BG
    bgf = bg.astype(jnp.float32)

    i_nb = jnp.sum(jnp.where(bg, 0.0, p * tf))
    t_nb = jnp.sum(jnp.where(bg, 0.0, p + tf))

    rows, cols = bgf.shape
    jj = lax.broadcasted_iota(jnp.int32, (cols, cols), 0)
    kk = lax.broadcasted_iota(jnp.int32, (cols, cols), 1)
    tri_inc = (jj <= kk).astype(jnp.float32)
    cs_in = jnp.dot(bgf, tri_inc, preferred_element_type=jnp.float32)
    row_tot = cs_in[:, cols - 1:cols]
    ii = lax.broadcasted_iota(jnp.int32, (rows, rows), 0)
    ll = lax.broadcasted_iota(jnp.int32, (rows, rows), 1)
    tri_lo = (ll < ii).astype(jnp.float32)
    r_pref = jnp.dot(tri_lo, row_tot, preferred_element_type=jnp.float32)
    rank_ex = r_pref + (cs_in - bgf)

    offset = cnt_ref[0].astype(jnp.float32)
    include = bg & (offset + rank_ex < _BUDGET)
    s_bg = jnp.sum(jnp.where(include, p, 0.0))
    n_inc = jnp.sum(include.astype(jnp.float32))

    acc_ref[0] += i_nb + jnp.float32(_BG) * s_bg
    acc_ref[1] += t_nb + s_bg + jnp.float32(_BG) * n_inc
    cnt_ref[0] += jnp.sum(bg.astype(jnp.int32))

    @pl.when(step == nsteps - 1)
    def _fin():
        out_ref[0, 0] = acc_ref[0]
        out_ref[0, 1] = acc_ref[1]
        out_ref[0, 2] = cnt_ref[0].astype(jnp.float32)


def _argmax_sc(inp_hbm, preds_hbm, buf0, buf1, pb0, pb1, si0, si1, so0, so1):
    cid = lax.axis_index("c")
    sid = lax.axis_index("s")
    wid = sid * 2 + cid
    r0 = wid * _RPW
    bufs = (buf0, buf1)
    pbs = (pb0, pb1)
    sis = (si0, si1)
    sos = (so0, so1)

    def in_copy(ci, k):
        rg = ci // _CGS
        cg = ci - rg * _CGS
        return pltpu.make_async_copy(
            inp_hbm.at[_TAIL_B, :, pl.ds(r0 + rg * _TR, _TR),
                       pl.ds(cg * _TC, _TC)],
            bufs[k], sis[k])

    def out_copy(ci, k):
        rg = ci // _CGS
        cg = ci - rg * _CGS
        return pltpu.make_async_copy(
            pbs[k],
            preds_hbm.at[pl.ds(r0 + rg * _TR, _TR), pl.ds(cg * _TC, _TC)],
            sos[k])

    in_copy(0, 0).start()
    in_copy(1, 1).start()

    def pair(i2, carry):
        for k in range(2):
            ci = i2 * 2 + k
            in_copy(ci, k).wait()

            @pl.when(i2 >= 1)
            def _wait_out():
                out_copy(ci - 2, k).wait()

            buf = bufs[k]
            pb = pbs[k]

            @plsc.parallel_loop(0, (_TR * _TC) // _LANES, unroll=4)
            def _grp(j):
                r = j // (_TC // _LANES)
                col = (j - r * (_TC // _LANES)) * _LANES
                s = pl.ds(col, _LANES)
                m = buf[0, r, s]
                idx = jnp.zeros((_LANES,), jnp.float32)
                for c in range(1, _NCLS):
                    v = buf[c, r, s]
                    gt = v > m
                    m = jnp.where(gt, v, m)
                    idx = jnp.where(gt, jnp.float32(c), idx)
                pb[r, s] = idx

            out_copy(ci, k).start()

            @pl.when(ci + 2 < _NCH)
            def _next_in():
                in_copy(ci + 2, k).start()

        return carry

    lax.fori_loop(0, _NCH // 2, pair, 0)
    out_copy(_NCH - 2, 0).wait()
    out_copy(_NCH - 1, 1).wait()


def _tail_gate(p_ref, t_ref, s_ref, out_ref):
    p = p_ref[...]
    t = t_ref[...]
    tf = t.astype(jnp.float32)
    bg = t == _BG
    bgf = bg.astype(jnp.float32)

    i_nb = jnp.sum(jnp.where(bg, 0.0, p * tf))
    t_nb = jnp.sum(jnp.where(bg, 0.0, p + tf))

    i_head = s_ref[0, 0]
    t_head = s_ref[0, 1]
    offset = s_ref[0, 2]

    cntf = jnp.sum(bgf)
    s_all = jnp.sum(jnp.where(bg, p, 0.0))
    all_in = offset + cntf <= _BUDGET

    def _finish(s_bg, n_inc):
        inter = i_head + i_nb + jnp.float32(_BG) * s_bg
        total = t_head + t_nb + s_bg + jnp.float32(_BG) * n_inc
        union = total - inter
        out_ref[0, 0] = 1.0 - (inter + 1.0) / (union + 1.0)

    @pl.when(all_in)
    def _fast():
        _finish(s_all, cntf)

    @pl.when(jnp.logical_not(all_in))
    def _ranked():
        rows, cols = bgf.shape
        jj = lax.broadcasted_iota(jnp.int32, (cols, cols), 0)
        kk = lax.broadcasted_iota(jnp.int32, (cols, cols), 1)
        tri_inc = (jj <= kk).astype(jnp.float32)
        cs_in = jnp.dot(bgf, tri_inc, preferred_element_type=jnp.float32)
        row_tot = cs_in[:, cols - 1:cols]
        ii = lax.broadcasted_iota(jnp.int32, (rows, rows), 0)
        ll = lax.broadcasted_iota(jnp.int32, (rows, rows), 1)
        tri_lo = (ll < ii).astype(jnp.float32)
        r_pref = jnp.dot(tri_lo, row_tot, preferred_element_type=jnp.float32)
        rank_ex = r_pref + (cs_in - bgf)
        include = bg & (offset + rank_ex < _BUDGET)
        s_bg = jnp.sum(jnp.where(include, p, 0.0))
        n_inc = jnp.sum(include.astype(jnp.float32))
        _finish(s_bg, n_inc)


def kernel(inputs, targets):
    sc_fn = pl.kernel(
        _argmax_sc,
        out_type=jax.ShapeDtypeStruct((_H, _W), jnp.float32),
        mesh=plsc.VectorSubcoreMesh(
            core_axis_name="c", subcore_axis_name="s",
            num_cores=2, num_subcores=16),
        scratch_types=[
            pltpu.VMEM((_NCLS, _TR, _TC), jnp.float32),
            pltpu.VMEM((_NCLS, _TR, _TC), jnp.float32),
            pltpu.VMEM((_TR, _TC), jnp.float32),
            pltpu.VMEM((_TR, _TC), jnp.float32),
            pltpu.SemaphoreType.DMA,
            pltpu.SemaphoreType.DMA,
            pltpu.SemaphoreType.DMA,
            pltpu.SemaphoreType.DMA,
        ],
    )
    preds_tail = sc_fn(inputs)

    stats = pl.pallas_call(
        _head_tc,
        grid=(_TAIL_B, _H // _ROWS),
        in_specs=[
            pl.BlockSpec((1, _NCLS, _ROWS, _W), lambda i, j: (i, 0, j, 0)),
            pl.BlockSpec((1, _ROWS, _W), lambda i, j: (i, j, 0)),
        ],
        out_specs=pl.BlockSpec(
            (1, 3), lambda i, j: (0, 0), memory_space=pltpu.SMEM),
        out_shape=jax.ShapeDtypeStruct((1, 3), jnp.float32),
        scratch_shapes=[
            pltpu.SMEM((2,), jnp.float32),
            pltpu.SMEM((1,), jnp.int32),
        ],
    )(inputs, targets)

    out = pl.pallas_call(
        _tail_gate,
        in_specs=[
            pl.BlockSpec((_H, _W), lambda: (0, 0)),
            pl.BlockSpec((_H, _W), lambda: (0, 0)),
            pl.BlockSpec((1, 3), lambda: (0, 0), memory_space=pltpu.SMEM),
        ],
        out_specs=pl.BlockSpec(
            (1, 1), lambda: (0, 0), memory_space=pltpu.SMEM),
        out_shape=jax.ShapeDtypeStruct((1, 1), jnp.float32),
    )(preds_tail, targets[_TAIL_B], stats)
    return out[0, 0]


# speedup vs baseline: 1.4944x; 1.4944x over previous
import jax
import jax.numpy as jnp
from jax.experimental import pallas as pl
from jax.experimental.pallas import tpu as pltpu

_NCLS = 31
_BG = 30
_BUDGET = 80000.0
_ROWS = 128


def _iou_kernel(x_ref, t_ref, out_ref, acc_ref, cnt_ref):
    b = pl.program_id(0)
    r = pl.program_id(1)
    nb = pl.num_programs(1)
    step = b * nb + r
    nsteps = pl.num_programs(0) * nb

    @pl.when(step == 0)
    def _init():
        acc_ref[0] = 0.0
        acc_ref[1] = 0.0
        cnt_ref[0] = 0

    x = x_ref[0]
    m = x[0]
    idx = jnp.zeros_like(m)
    for c in range(1, _NCLS):
        xc = x[c]
        gt = xc > m
        m = jnp.where(gt, xc, m)
        idx = jnp.where(gt, jnp.float32(c), idx)
    p = idx

    t = t_ref[0]
    tf = t.astype(jnp.float32)
    bg = t == _BG
    bgf = bg.astype(jnp.float32)

    i_nb = jnp.sum(jnp.where(bg, 0.0, p * tf))
    t_nb = jnp.sum(jnp.where(bg, 0.0, p + tf))

    rows, cols = bgf.shape
    jj = jax.lax.broadcasted_iota(jnp.int32, (cols, cols), 0)
    kk = jax.lax.broadcasted_iota(jnp.int32, (cols, cols), 1)
    tri_inc = (jj <= kk).astype(jnp.float32)
    cs_in = jnp.dot(bgf, tri_inc, preferred_element_type=jnp.float32)
    row_tot = cs_in[:, cols - 1:cols]
    ii = jax.lax.broadcasted_iota(jnp.int32, (rows, rows), 0)
    ll = jax.lax.broadcasted_iota(jnp.int32, (rows, rows), 1)
    tri_lo = (ll < ii).astype(jnp.float32)
    r_pref = jnp.dot(tri_lo, row_tot, preferred_element_type=jnp.float32)
    rank_ex = r_pref + (cs_in - bgf)

    offset = cnt_ref[0].astype(jnp.float32)
    include = bg & (offset + rank_ex < _BUDGET)
    s_bg = jnp.sum(jnp.where(include, p, 0.0))
    n_inc = jnp.sum(include.astype(jnp.float32))

    acc_ref[0] += i_nb + jnp.float32(_BG) * s_bg
    acc_ref[1] += t_nb + s_bg + jnp.float32(_BG) * n_inc
    cnt_ref[0] += jnp.sum(bg.astype(jnp.int32))

    @pl.when(step == nsteps - 1)
    def _fin():
        inter = acc_ref[0]
        total = acc_ref[1]
        union = total - inter
        out_ref[0, 0] = 1.0 - (inter + 1.0) / (union + 1.0)


def kernel(inputs, targets):
    b, ncls, h, w = inputs.shape
    nb = h // _ROWS
    out = pl.pallas_call(
        _iou_kernel,
        grid=(b, nb),
        in_specs=[
            pl.BlockSpec((1, ncls, _ROWS, w), lambda i, j: (i, 0, j, 0)),
            pl.BlockSpec((1, _ROWS, w), lambda i, j: (i, j, 0)),
        ],
        out_specs=pl.BlockSpec(
            (1, 1), lambda i, j: (0, 0), memory_space=pltpu.SMEM),
        out_shape=jax.ShapeDtypeStruct((1, 1), jnp.float32),
        scratch_shapes=[
            pltpu.SMEM((2,), jnp.float32),
            pltpu.SMEM((1,), jnp.int32),
        ],
    )(inputs, targets)
    return out[0, 0]
